# Initial kernel scaffold; baseline (speedup 1.0000x reference)
import jax
import jax.numpy as jnp
from jax.experimental import pallas as pl


def kernel(x, edge_index, edge_attr, batch, conc, temp, pco2, params):
    raise NotImplementedError("write your pallas kernel here")


# speedup vs baseline: 2.3722x; 2.3722x over previous
import functools

import jax
import jax.numpy as jnp
from jax import lax
from jax.experimental import pallas as pl
from jax.experimental.pallas import tpu as pltpu
from jax.experimental.pallas import tpu_sc as plsc

N_NODES = 10000
N_PAD = 10240
N_EDGES = 320000
E_PAD = 321536
CHUNK = 128
CHUNKS_PER_TILE = 157
ROWS_PER_TILE = 640
H = 256
HALF = 128
NC = 2
NS = 16


def _sc_mesh():
    return plsc.VectorSubcoreMesh(core_axis_name="c", subcore_axis_name="s",
                                  num_cores=NC, num_subcores=NS)


def _make_sc_edge_kernel():
    scratch = [
        pltpu.VMEM((CHUNK, HALF), jnp.float32),
        pltpu.VMEM((CHUNK, HALF), jnp.float32),
        pltpu.VMEM((CHUNK,), jnp.int32),
        pltpu.VMEM((CHUNK,), jnp.int32),
        pltpu.VMEM_SHARED((N_PAD, HALF), jnp.float32),
    ]

    def body(a_hbm, e_hbm, src_hbm, dst_hbm, r_out,
             e_buf, g_buf, src_v, dst_v, acc):
        c = lax.axis_index("c")
        s = lax.axis_index("s")
        zero16 = jnp.zeros((16,), jnp.float32)

        def zrow(r, _):
            for j in range(HALF // 16):
                e_buf[r, pl.ds(j * 16, 16)] = zero16
            return 0
        lax.fori_loop(0, CHUNK, zrow, 0)
        for rep in range(ROWS_PER_TILE // CHUNK):
            pltpu.sync_copy(
                e_buf, acc.at[pl.ds(s * ROWS_PER_TILE + rep * CHUNK, CHUNK)])
        plsc.subcore_barrier()

        row_off = c * N_PAD

        def chunk_body(k, _):
            base = pl.multiple_of(s * (CHUNKS_PER_TILE * CHUNK) + k * CHUNK, 8)
            pltpu.sync_copy(src_hbm.at[pl.ds(base, CHUNK)], src_v)
            pltpu.sync_copy(dst_hbm.at[pl.ds(base, CHUNK)], dst_v)
            for j in range(CHUNK // 16):
                sl = pl.ds(j * 16, 16)
                src_v[sl] = src_v[sl] + row_off
            pltpu.sync_copy(
                e_hbm.at[pl.ds(c * E_PAD + base, CHUNK)], e_buf)
            pltpu.sync_copy(a_hbm.at[src_v], g_buf)

            def row(r, _):
                for j in range(HALF // 16):
                    sl = pl.ds(j * 16, 16)
                    g_buf[r, sl] = jnp.maximum(g_buf[r, sl] + e_buf[r, sl],
                                               0.0)
                return 0
            lax.fori_loop(0, CHUNK, row, 0)

            pltpu.sync_copy(g_buf, acc.at[dst_v], add=True)
            return 0

        lax.fori_loop(0, CHUNKS_PER_TILE, chunk_body, 0)
        plsc.subcore_barrier()

        rb = s * ROWS_PER_TILE
        pltpu.sync_copy(acc.at[pl.ds(rb, ROWS_PER_TILE)],
                        r_out.at[pl.ds(c * N_PAD + rb, ROWS_PER_TILE)])

    return pl.kernel(
        body,
        out_type=jax.ShapeDtypeStruct((NC * N_PAD, HALF), jnp.float32),
        mesh=_sc_mesh(), scratch_types=scratch)


def _make_sc_cnt_kernel():
    scratch = [
        pltpu.VMEM((CHUNK, HALF), jnp.float32),
        pltpu.VMEM((CHUNK,), jnp.int32),
        pltpu.VMEM_SHARED((N_PAD, HALF), jnp.float32),
    ]

    def body(dst_hbm, cnt_out, cbuf, dst_v, cnt_acc):
        s = lax.axis_index("s")

        def fill(val):
            v = jnp.full((16,), val, jnp.float32)
            def frow(r, _):
                for j in range(HALF // 16):
                    cbuf[r, pl.ds(j * 16, 16)] = v
                return 0
            lax.fori_loop(0, CHUNK, frow, 0)

        fill(0.0)
        for rep in range(ROWS_PER_TILE // CHUNK):
            pltpu.sync_copy(
                cbuf, cnt_acc.at[pl.ds(s * ROWS_PER_TILE + rep * CHUNK, CHUNK)])
        fill(1.0)
        plsc.subcore_barrier()

        def chunk_body(k, _):
            base = pl.multiple_of(s * (CHUNKS_PER_TILE * CHUNK) + k * CHUNK, 8)
            pltpu.sync_copy(dst_hbm.at[pl.ds(base, CHUNK)], dst_v)
            pltpu.sync_copy(cbuf, cnt_acc.at[dst_v], add=True)
            return 0

        lax.fori_loop(0, CHUNKS_PER_TILE, chunk_body, 0)
        plsc.subcore_barrier()

        c = lax.axis_index("c")
        rb = s * ROWS_PER_TILE
        pltpu.sync_copy(cnt_acc.at[pl.ds(rb, ROWS_PER_TILE)],
                        cnt_out.at[pl.ds(c * N_PAD + rb, ROWS_PER_TILE)])

    return pl.kernel(
        body,
        out_type=jax.ShapeDtypeStruct((NC * N_PAD, HALF), jnp.float32),
        mesh=_sc_mesh(), scratch_types=scratch)


def _mm_bias(x, wT, b):
    m, k = x.shape
    n = wT.shape[1]
    blk = 512
    b2 = b.reshape(1, n)

    def body(x_ref, w_ref, b_ref, o_ref):
        o_ref[...] = jnp.dot(x_ref[...], w_ref[...],
                             preferred_element_type=jnp.float32) + b_ref[...]

    return pl.pallas_call(
        body,
        grid=(m // blk,),
        in_specs=[pl.BlockSpec((blk, k), lambda i: (i, 0)),
                  pl.BlockSpec((k, n), lambda i: (0, 0)),
                  pl.BlockSpec((1, n), lambda i: (0, 0))],
        out_specs=pl.BlockSpec((blk, n), lambda i: (i, 0)),
        out_shape=jax.ShapeDtypeStruct((m, n), jnp.float32),
    )(x, wT, b2)


def _a_split(x_t, w1aT_split):
    blk = 512

    def body(x_ref, w_ref, o_ref):
        o_ref[0] = jnp.dot(x_ref[...], w_ref[0],
                           preferred_element_type=jnp.float32)

    return pl.pallas_call(
        body,
        grid=(NC, N_PAD // blk),
        in_specs=[pl.BlockSpec((blk, H), lambda c, i: (i, 0)),
                  pl.BlockSpec((1, H, HALF), lambda c, i: (c, 0, 0))],
        out_specs=pl.BlockSpec((1, blk, HALF), lambda c, i: (c, i, 0)),
        out_shape=jax.ShapeDtypeStruct((NC, N_PAD, HALF), jnp.float32),
    )(x_t, w1aT_split)


def _e_all(ea_pad, WeT_all, ce0_all):
    blk = 2048

    def body(ea_ref, w_ref, b_ref, o_ref):
        o_ref[0, 0] = jnp.dot(ea_ref[...], w_ref[0, 0],
                              preferred_element_type=jnp.float32) + b_ref[0, 0]

    return pl.pallas_call(
        body,
        grid=(3, NC, E_PAD // blk),
        in_specs=[pl.BlockSpec((blk, 16), lambda l, c, i: (i, 0)),
                  pl.BlockSpec((1, 1, 16, HALF), lambda l, c, i: (l, c, 0, 0)),
                  pl.BlockSpec((1, 1, 1, HALF), lambda l, c, i: (l, c, 0, 0))],
        out_specs=pl.BlockSpec((1, 1, blk, HALF), lambda l, c, i: (l, c, i, 0)),
        out_shape=jax.ShapeDtypeStruct((3, NC, E_PAD, HALF), jnp.float32),
    )(ea_pad, WeT_all, ce0_all)


def _post_layer(r0, r1, a0, a1, x_t, cnt_e, h_prev, w2T, b2, c0,
                uw1aT, uw1bT, uw2T, ub1, ub2, ln_g, ln_b, skip):
    blk = 512

    def body(r0_r, r1_r, a0_r, a1_r, xt_r, cnt_r, hp_r, w2_r, b2_r, c0_r,
             u1a_r, u1b_r, u2_r, ub1_r, ub2_r, g_r, bln_r, sk_r, o_ref):
        r_full = jnp.concatenate([r0_r[...], r1_r[...]], axis=-1)
        a_full = jnp.concatenate([a0_r[...], a1_r[...]], axis=-1)
        r_full = r_full + jnp.maximum(a_full + c0_r[...], 0.0)
        cnt = cnt_r[...] + 1.0
        aggr = jnp.dot(r_full, w2_r[...],
                       preferred_element_type=jnp.float32) / cnt + b2_r[...]
        u = jnp.maximum(
            jnp.dot(aggr, u1a_r[...], preferred_element_type=jnp.float32)
            + jnp.dot(xt_r[...], u1b_r[...], preferred_element_type=jnp.float32)
            + ub1_r[...], 0.0)
        u = jnp.dot(u, u2_r[...], preferred_element_type=jnp.float32) + ub2_r[...]
        mu = jnp.mean(u, axis=-1, keepdims=True)
        var = jnp.mean((u - mu) ** 2, axis=-1, keepdims=True)
        u = (u - mu) / jnp.sqrt(var + 1e-5) * g_r[...] + bln_r[...]
        o_ref[...] = u + sk_r[...] * hp_r[...]

    full = lambda shape: pl.BlockSpec(shape, lambda i: (0, 0))
    return pl.pallas_call(
        body,
        grid=(N_PAD // blk,),
        in_specs=[pl.BlockSpec((blk, HALF), lambda i: (i, 0)),
                  pl.BlockSpec((blk, HALF), lambda i: (i, 0)),
                  pl.BlockSpec((blk, HALF), lambda i: (i, 0)),
                  pl.BlockSpec((blk, HALF), lambda i: (i, 0)),
                  pl.BlockSpec((blk, H), lambda i: (i, 0)),
                  pl.BlockSpec((blk, 1), lambda i: (i, 0)),
                  pl.BlockSpec((blk, H), lambda i: (i, 0)),
                  full((H, H)), full((1, H)), full((1, H)),
                  full((H, H)), full((H, H)), full((H, H)),
                  full((1, H)), full((1, H)), full((1, H)), full((1, H)),
                  full((1, 1))],
        out_specs=pl.BlockSpec((blk, H), lambda i: (i, 0)),
        out_shape=jax.ShapeDtypeStruct((N_PAD, H), jnp.float32),
    )(r0, r1, a0, a1, x_t, cnt_e, h_prev, w2T, b2, c0,
      uw1aT, uw1bT, uw2T, ub1, ub2, ln_g, ln_b, skip)


def _pool(h, batch_f, n_b):
    blk = 512

    def body(h_ref, b_ref, s_ref, c_ref):
        i = pl.program_id(0)
        iota = lax.broadcasted_iota(jnp.int32, (1, n_b), 1).astype(jnp.float32)
        onehot = (b_ref[...] == iota).astype(jnp.float32)
        s_blk = lax.dot_general(onehot, h_ref[...],
                                (((0,), (0,)), ((), ())),
                                preferred_element_type=jnp.float32)
        c_blk = lax.dot_general(onehot, jnp.ones((blk, 1), jnp.float32),
                                (((0,), (0,)), ((), ())),
                                preferred_element_type=jnp.float32)

        @pl.when(i == 0)
        def _():
            s_ref[...] = jnp.zeros_like(s_ref)
            c_ref[...] = jnp.zeros_like(c_ref)
        s_ref[...] += s_blk
        c_ref[...] += c_blk

    return pl.pallas_call(
        body,
        grid=(N_PAD // blk,),
        in_specs=[pl.BlockSpec((blk, H), lambda i: (i, 0)),
                  pl.BlockSpec((blk, 1), lambda i: (i, 0))],
        out_specs=[pl.BlockSpec((n_b, H), lambda i: (0, 0)),
                   pl.BlockSpec((n_b, 1), lambda i: (0, 0))],
        out_shape=[jax.ShapeDtypeStruct((n_b, H), jnp.float32),
                   jax.ShapeDtypeStruct((n_b, 1), jnp.float32)],
    )(h, batch_f)


def _head(s, cnt_b, add8, w0gT, w0aT, b0, w1T, b1, w2T, b2c, w3T, b3,
          lng, lnb, n_b):
    def ln(z, g, b):
        mu = jnp.mean(z, axis=-1, keepdims=True)
        var = jnp.mean((z - mu) ** 2, axis=-1, keepdims=True)
        return (z - mu) / jnp.sqrt(var + 1e-5) * g + b

    def body(s_r, c_r, a_r, w0g_r, w0a_r, b0_r, w1_r, b1_r, w2_r, b2_r,
             w3_r, b3_r, lng_r, lnb_r, o_ref):
        g = s_r[...] / jnp.maximum(c_r[...], 1.0)
        z = (jnp.dot(g, w0g_r[...], preferred_element_type=jnp.float32)
             + jnp.dot(a_r[...], w0a_r[...], preferred_element_type=jnp.float32)
             + b0_r[...])
        z = jnp.maximum(ln(z, lng_r[0, 0:1, :], lnb_r[0, 0:1, :]), 0.0)
        z = jnp.dot(z, w1_r[...], preferred_element_type=jnp.float32) + b1_r[...]
        z = jnp.maximum(ln(z, lng_r[1, 0:1, :], lnb_r[1, 0:1, :]), 0.0)
        z = jnp.dot(z, w2_r[...], preferred_element_type=jnp.float32) + b2_r[...]
        z = jnp.maximum(ln(z, lng_r[2, 0:1, :], lnb_r[2, 0:1, :]), 0.0)
        z = jnp.dot(z, w3_r[...], preferred_element_type=jnp.float32) + b3_r[...]
        o_ref[...] = jnp.maximum(z, 0.0) + jnp.log1p(jnp.exp(-jnp.abs(z)))

    full = lambda shape: pl.BlockSpec(shape, lambda: tuple(0 for _ in shape))
    return pl.pallas_call(
        body,
        in_specs=[full((n_b, H)), full((n_b, 1)), full((n_b, 8)),
                  full((H, H)), full((8, H)), full((1, H)),
                  full((H, H)), full((1, H)), full((H, H)), full((1, H)),
                  full((H, 8)), full((1, 8)), full((3, 1, H)), full((3, 1, H))],
        out_specs=full((n_b, 8)),
        out_shape=jax.ShapeDtypeStruct((n_b, 8), jnp.float32),
    )(s, cnt_b, add8, w0gT, w0aT, b0, w1T, b1, w2T, b2c, w3T, b3, lng, lnb)


def kernel(x, edge_index, edge_attr, batch, conc, temp, pco2, params):
    f32 = jnp.float32
    n_b = 128

    x_pad = jnp.pad(x, ((0, N_PAD - N_NODES), (0, 0)))
    src = jnp.pad(edge_index[0], (0, E_PAD - N_EDGES),
                  constant_values=N_PAD - 1)
    dst = jnp.pad(edge_index[1], (0, E_PAD - N_EDGES),
                  constant_values=N_PAD - 1)
    ea_pad = jnp.pad(edge_attr, ((0, E_PAD - N_EDGES), (0, 0)))
    batch_f = jnp.pad(batch.astype(f32).reshape(-1, 1),
                      ((0, N_PAD - N_NODES), (0, 0)), constant_values=-1.0)
    add8 = jnp.pad(jnp.stack([conc, temp, pco2], axis=1).astype(f32),
                   ((0, 0), (0, 5)))

    Ls = params['layers']
    WeT_all, ce0_all = [], []
    prep = []
    for i in range(3):
        p = Ls[i]
        w1a = p['msg_w1'][:, :H]
        w1b = p['msg_w1'][:, H:]
        We = w1b @ p['lin_edge_w']
        ce0 = w1b @ p['lin_edge_b'] + p['msg_b1']
        WeT_all.append(We.T.reshape(16, NC, HALF).transpose(1, 0, 2))
        ce0_all.append(ce0.reshape(NC, 1, HALF))
        prep.append(dict(
            lwT=p['lin_node_w'].T, lb=p['lin_node_b'],
            w1aT_split=w1a.T.reshape(H, NC, HALF).transpose(1, 0, 2),
            c0=(We.sum(axis=1) + ce0).reshape(1, H),
            w2T=p['msg_w2'].T, b2=p['msg_b2'].reshape(1, H),
            uw1aT=p['upd_w1'][:, :H].T, uw1bT=p['upd_w1'][:, H:].T,
            uw2T=p['upd_w2'].T, ub1=p['upd_b1'].reshape(1, H),
            ub2=p['upd_b2'].reshape(1, H),
            ln_g=params['norm_g'][i].reshape(1, H),
            ln_b=params['norm_b'][i].reshape(1, H),
        ))
    WeT_all = jnp.stack(WeT_all)
    ce0_all = jnp.stack(ce0_all)
    skips = jnp.maximum(params['skip_weights'], 0.0)

    E_all = _e_all(ea_pad, WeT_all, ce0_all)

    sc_edge = _make_sc_edge_kernel()
    cnt_sc = _make_sc_cnt_kernel()(dst)
    cnt_e = cnt_sc[:N_PAD, :1]

    h = x_pad
    for i in range(3):
        pr = prep[i]
        x_t = _mm_bias(h, pr['lwT'], pr['lb'])
        a_sp = _a_split(x_t, pr['w1aT_split'])
        a_tab = a_sp.reshape(NC * N_PAD, HALF)
        e_l = E_all[i].reshape(NC * E_PAD, HALF)
        r_sc = sc_edge(a_tab, e_l, src, dst)
        skip = (skips[i - 1].reshape(1, 1) if i > 0
                else jnp.zeros((1, 1), f32))
        h_prev = h if i > 0 else x_t
        h = _post_layer(
            r_sc[:N_PAD], r_sc[N_PAD:], a_sp[0], a_sp[1], x_t, cnt_e, h_prev,
            pr['w2T'], pr['b2'], pr['c0'], pr['uw1aT'], pr['uw1bT'],
            pr['uw2T'], pr['ub1'], pr['ub2'], pr['ln_g'], pr['ln_b'], skip)

    s, cnt_b = _pool(h, batch_f, n_b)

    fc = params['fc']
    out8 = _head(
        s, cnt_b, add8,
        fc['ws'][0][:, :H].T, jnp.pad(fc['ws'][0][:, H:], ((0, 0), (0, 5))).T,
        fc['bs'][0].reshape(1, H),
        fc['ws'][1].T, fc['bs'][1].reshape(1, H),
        fc['ws'][2].T, fc['bs'][2].reshape(1, H),
        jnp.pad(fc['ws'][3], ((0, 7), (0, 0))).T,
        jnp.pad(fc['bs'][3], (0, 7)).reshape(1, 8),
        jnp.stack([g.reshape(1, H) for g in fc['ln_g']]),
        jnp.stack([b.reshape(1, H) for b in fc['ln_b']]),
        n_b)
    return out8[:, :1]


# speedup vs baseline: 2.6658x; 1.1238x over previous
import functools

import jax
import jax.numpy as jnp
from jax import lax
from jax.experimental import pallas as pl
from jax.experimental.pallas import tpu as pltpu
from jax.experimental.pallas import tpu_sc as plsc

N_NODES = 10000
N_PAD = 10240
N_EDGES = 320000
E_PAD = 327680
ECHUNK = 64
ECHUNKS = 320
SUPER = 16
NSUPER = ECHUNKS // SUPER
IDX_ROWS = E_PAD // ECHUNK
CCHUNK = 128
CCHUNKS = E_PAD // (16 * CCHUNK)
ROWS_PER_TILE = 640
H = 256
HALF = 128
NC = 2
NS = 16


def _sc_mesh():
    return plsc.VectorSubcoreMesh(core_axis_name="c", subcore_axis_name="s",
                                  num_cores=NC, num_subcores=NS)


def _make_sc_edge_kernel():
    f32, i32 = jnp.float32, jnp.int32
    scratch = [
        pltpu.VMEM((ECHUNK, HALF), f32), pltpu.VMEM((ECHUNK, HALF), f32),
        pltpu.VMEM((ECHUNK, HALF), f32), pltpu.VMEM((ECHUNK, HALF), f32),
        pltpu.VMEM((SUPER, ECHUNK), i32), pltpu.VMEM((SUPER, ECHUNK), i32),
        pltpu.VMEM((SUPER, ECHUNK), i32), pltpu.VMEM((SUPER, ECHUNK), i32),
        pltpu.VMEM((ECHUNK,), i32), pltpu.VMEM((ECHUNK,), i32),
        pltpu.VMEM((ECHUNK,), i32), pltpu.VMEM((ECHUNK,), i32),
        pltpu.VMEM_SHARED((N_PAD, HALF), f32),
    ] + [pltpu.SemaphoreType.DMA] * 8

    def body(a_hbm, e_hbm, src2_hbm, dst2_hbm, r_out,
             e0, e1, g0, g1, ssb0, ssb1, dsb0, dsb1, sv0, sv1, dv0, dv1,
             acc, ldg0, ldg1, lde0, lde1, sc0, sc1, ix0, ix1):
        e_b, g_b = (e0, e1), (g0, g1)
        ssb, dsb = (ssb0, ssb1), (dsb0, dsb1)
        sv, dv = (sv0, sv1), (dv0, dv1)
        ldg, lde, scs, ixs = (ldg0, ldg1), (lde0, lde1), (sc0, sc1), (ix0, ix1)
        c = lax.axis_index("c")
        s = lax.axis_index("s")
        row_off = c * N_PAD
        zero16 = jnp.zeros((16,), jnp.float32)

        def zrow(r, _):
            for j in range(HALF // 16):
                e0[r, pl.ds(j * 16, 16)] = zero16
            return 0
        lax.fori_loop(0, ECHUNK, zrow, 0)
        for rep in range(ROWS_PER_TILE // ECHUNK):
            pltpu.sync_copy(
                e0, acc.at[pl.ds(s * ROWS_PER_TILE + rep * ECHUNK, ECHUNK)])
        plsc.subcore_barrier()

        def issue_idx(sbi, q):
            r0 = s * ECHUNKS + sbi * SUPER
            pltpu.async_copy(src2_hbm.at[pl.ds(r0, SUPER)], ssb[q], ixs[q])
            pltpu.async_copy(dst2_hbm.at[pl.ds(r0, SUPER)], dsb[q], ixs[q])

        def wait_idx(q):
            pltpu.make_async_copy(src2_hbm.at[pl.ds(0, SUPER)], ssb[q],
                                  ixs[q]).wait()
            pltpu.make_async_copy(dst2_hbm.at[pl.ds(0, SUPER)], dsb[q],
                                  ixs[q]).wait()

        def fill_regs(kk, my_sv, my_dv):
            row1 = kk & (SUPER - 1)
            q1 = (kk // SUPER) & 1
            for q in range(2):
                @pl.when(q1 == q)
                def _():
                    for i in range(ECHUNK // 16):
                        sl = pl.ds(i * 16, 16)
                        my_sv[sl] = ssb[q][row1, sl] + row_off
                        my_dv[sl] = dsb[q][row1, sl]

        def issue_chunk(kk, my_sv, my_e, my_g, my_ldg, my_lde):
            base = s * (ECHUNKS * ECHUNK) + kk * ECHUNK
            pltpu.async_copy(a_hbm.at[my_sv], my_g, my_ldg)
            pltpu.async_copy(e_hbm.at[pl.ds(c * E_PAD + base, ECHUNK)],
                             my_e, my_lde)

        issue_idx(0, 0)
        issue_idx(1, 1)
        wait_idx(0)
        fill_regs(0, sv0, dv0)
        issue_chunk(0, sv0, e0, g0, ldg0, lde0)

        def step(k, my_e, my_g, my_sv, my_dv, my_ldg, my_lde, my_sc,
                 nx_e, nx_g, nx_sv, nx_dv, nx_ldg, nx_lde, nx_sc):
            @pl.when(k < ECHUNKS - 1)
            def _():
                k1 = k + 1
                row1 = k1 & (SUPER - 1)
                sb1 = k1 // SUPER
                q1 = sb1 & 1

                @pl.when(row1 == 0)
                def _():
                    for q in range(2):
                        @pl.when(q1 == q)
                        def _():
                            wait_idx(q)
                            @pl.when(sb1 + 1 < NSUPER)
                            def _():
                                issue_idx(sb1 + 1, 1 - q)

                @pl.when(k >= 1)
                def _():
                    pltpu.make_async_copy(nx_g, acc.at[nx_dv], nx_sc).wait()
                fill_regs(k1, nx_sv, nx_dv)
                issue_chunk(k1, nx_sv, nx_e, nx_g, nx_ldg, nx_lde)

            pltpu.make_async_copy(a_hbm.at[my_sv], my_g, my_ldg).wait()
            pltpu.make_async_copy(e_hbm.at[pl.ds(0, ECHUNK)], my_e,
                                  my_lde).wait()

            def rowf(r, _):
                for j in range(HALF // 16):
                    sl = pl.ds(j * 16, 16)
                    my_g[r, sl] = jnp.maximum(my_g[r, sl] + my_e[r, sl], 0.0)
                return 0
            lax.fori_loop(0, ECHUNK, rowf, 0)

            pltpu.async_copy(my_g, acc.at[my_dv], my_sc, add=True)

        def loop(t, _):
            k = t * 2
            step(k, e0, g0, sv0, dv0, ldg0, lde0, sc0,
                 e1, g1, sv1, dv1, ldg1, lde1, sc1)
            step(k + 1, e1, g1, sv1, dv1, ldg1, lde1, sc1,
                 e0, g0, sv0, dv0, ldg0, lde0, sc0)
            return 0

        lax.fori_loop(0, ECHUNKS // 2, loop, 0)
        pltpu.make_async_copy(g0, acc.at[dv0], sc0).wait()
        pltpu.make_async_copy(g1, acc.at[dv1], sc1).wait()
        plsc.subcore_barrier()

        rb = s * ROWS_PER_TILE
        pltpu.sync_copy(acc.at[pl.ds(rb, ROWS_PER_TILE)],
                        r_out.at[pl.ds(c * N_PAD + rb, ROWS_PER_TILE)])

    return pl.kernel(
        body,
        out_type=jax.ShapeDtypeStruct((NC * N_PAD, HALF), jnp.float32),
        mesh=_sc_mesh(), scratch_types=scratch)


def _make_sc_cnt_kernel():
    scratch = [
        pltpu.VMEM((CCHUNK, HALF), jnp.float32),
        pltpu.VMEM((CCHUNK,), jnp.int32),
        pltpu.VMEM_SHARED((N_PAD, HALF), jnp.float32),
    ]

    def body(dst_hbm, cnt_out, cbuf, dst_v, cnt_acc):
        s = lax.axis_index("s")

        def fill(val):
            v = jnp.full((16,), val, jnp.float32)
            def frow(r, _):
                for j in range(HALF // 16):
                    cbuf[r, pl.ds(j * 16, 16)] = v
                return 0
            lax.fori_loop(0, CCHUNK, frow, 0)

        fill(0.0)
        for rep in range(ROWS_PER_TILE // CCHUNK):
            pltpu.sync_copy(
                cbuf, cnt_acc.at[pl.ds(s * ROWS_PER_TILE + rep * CCHUNK, CCHUNK)])
        fill(1.0)
        plsc.subcore_barrier()

        def chunk_body(k, _):
            base = pl.multiple_of(s * (CCHUNKS * CCHUNK) + k * CCHUNK, 8)
            pltpu.sync_copy(dst_hbm.at[pl.ds(base, CCHUNK)], dst_v)
            pltpu.sync_copy(cbuf, cnt_acc.at[dst_v], add=True)
            return 0

        lax.fori_loop(0, CCHUNKS, chunk_body, 0)
        plsc.subcore_barrier()

        c = lax.axis_index("c")
        rb = s * ROWS_PER_TILE
        pltpu.sync_copy(cnt_acc.at[pl.ds(rb, ROWS_PER_TILE)],
                        cnt_out.at[pl.ds(c * N_PAD + rb, ROWS_PER_TILE)])

    return pl.kernel(
        body,
        out_type=jax.ShapeDtypeStruct((NC * N_PAD, HALF), jnp.float32),
        mesh=_sc_mesh(), scratch_types=scratch)


def _mm_bias(x, wT, b):
    m, k = x.shape
    n = wT.shape[1]
    blk = 512
    b2 = b.reshape(1, n)

    def body(x_ref, w_ref, b_ref, o_ref):
        o_ref[...] = jnp.dot(x_ref[...], w_ref[...],
                             preferred_element_type=jnp.float32) + b_ref[...]

    return pl.pallas_call(
        body,
        grid=(m // blk,),
        in_specs=[pl.BlockSpec((blk, k), lambda i: (i, 0)),
                  pl.BlockSpec((k, n), lambda i: (0, 0)),
                  pl.BlockSpec((1, n), lambda i: (0, 0))],
        out_specs=pl.BlockSpec((blk, n), lambda i: (i, 0)),
        out_shape=jax.ShapeDtypeStruct((m, n), jnp.float32),
    )(x, wT, b2)


def _a_split(x_t, w1aT_split):
    blk = 512

    def body(x_ref, w_ref, o_ref):
        o_ref[0] = jnp.dot(x_ref[...], w_ref[0],
                           preferred_element_type=jnp.float32)

    return pl.pallas_call(
        body,
        grid=(NC, N_PAD // blk),
        in_specs=[pl.BlockSpec((blk, H), lambda c, i: (i, 0)),
                  pl.BlockSpec((1, H, HALF), lambda c, i: (c, 0, 0))],
        out_specs=pl.BlockSpec((1, blk, HALF), lambda c, i: (c, i, 0)),
        out_shape=jax.ShapeDtypeStruct((NC, N_PAD, HALF), jnp.float32),
    )(x_t, w1aT_split)


def _e_all(ea_pad, WeT_all, ce0_all):
    blk = 2048

    def body(ea_ref, w_ref, b_ref, o_ref):
        o_ref[0, 0] = jnp.dot(ea_ref[...], w_ref[0, 0],
                              preferred_element_type=jnp.float32) + b_ref[0, 0]

    return pl.pallas_call(
        body,
        grid=(3, NC, E_PAD // blk),
        in_specs=[pl.BlockSpec((blk, 16), lambda l, c, i: (i, 0)),
                  pl.BlockSpec((1, 1, 16, HALF), lambda l, c, i: (l, c, 0, 0)),
                  pl.BlockSpec((1, 1, 1, HALF), lambda l, c, i: (l, c, 0, 0))],
        out_specs=pl.BlockSpec((1, 1, blk, HALF), lambda l, c, i: (l, c, i, 0)),
        out_shape=jax.ShapeDtypeStruct((3, NC, E_PAD, HALF), jnp.float32),
    )(ea_pad, WeT_all, ce0_all)


def _post_layer(r0, r1, a0, a1, x_t, cnt_e, h_prev, w2T, b2, c0,
                uw1aT, uw1bT, uw2T, ub1, ub2, ln_g, ln_b, skip):
    blk = 512

    def body(r0_r, r1_r, a0_r, a1_r, xt_r, cnt_r, hp_r, w2_r, b2_r, c0_r,
             u1a_r, u1b_r, u2_r, ub1_r, ub2_r, g_r, bln_r, sk_r, o_ref):
        r_full = jnp.concatenate([r0_r[...], r1_r[...]], axis=-1)
        a_full = jnp.concatenate([a0_r[...], a1_r[...]], axis=-1)
        r_full = r_full + jnp.maximum(a_full + c0_r[...], 0.0)
        cnt = cnt_r[...] + 1.0
        aggr = jnp.dot(r_full, w2_r[...],
                       preferred_element_type=jnp.float32) / cnt + b2_r[...]
        u = jnp.maximum(
            jnp.dot(aggr, u1a_r[...], preferred_element_type=jnp.float32)
            + jnp.dot(xt_r[...], u1b_r[...], preferred_element_type=jnp.float32)
            + ub1_r[...], 0.0)
        u = jnp.dot(u, u2_r[...], preferred_element_type=jnp.float32) + ub2_r[...]
        mu = jnp.mean(u, axis=-1, keepdims=True)
        var = jnp.mean((u - mu) ** 2, axis=-1, keepdims=True)
        u = (u - mu) / jnp.sqrt(var + 1e-5) * g_r[...] + bln_r[...]
        o_ref[...] = u + sk_r[...] * hp_r[...]

    full = lambda shape: pl.BlockSpec(shape, lambda i: (0, 0))
    return pl.pallas_call(
        body,
        grid=(N_PAD // blk,),
        in_specs=[pl.BlockSpec((blk, HALF), lambda i: (i, 0)),
                  pl.BlockSpec((blk, HALF), lambda i: (i, 0)),
                  pl.BlockSpec((blk, HALF), lambda i: (i, 0)),
                  pl.BlockSpec((blk, HALF), lambda i: (i, 0)),
                  pl.BlockSpec((blk, H), lambda i: (i, 0)),
                  pl.BlockSpec((blk, 1), lambda i: (i, 0)),
                  pl.BlockSpec((blk, H), lambda i: (i, 0)),
                  full((H, H)), full((1, H)), full((1, H)),
                  full((H, H)), full((H, H)), full((H, H)),
                  full((1, H)), full((1, H)), full((1, H)), full((1, H)),
                  full((1, 1))],
        out_specs=pl.BlockSpec((blk, H), lambda i: (i, 0)),
        out_shape=jax.ShapeDtypeStruct((N_PAD, H), jnp.float32),
    )(r0, r1, a0, a1, x_t, cnt_e, h_prev, w2T, b2, c0,
      uw1aT, uw1bT, uw2T, ub1, ub2, ln_g, ln_b, skip)


def _pool(h, batch_f, n_b):
    blk = 512

    def body(h_ref, b_ref, s_ref, c_ref):
        i = pl.program_id(0)
        iota = lax.broadcasted_iota(jnp.int32, (1, n_b), 1).astype(jnp.float32)
        onehot = (b_ref[...] == iota).astype(jnp.float32)
        s_blk = lax.dot_general(onehot, h_ref[...],
                                (((0,), (0,)), ((), ())),
                                preferred_element_type=jnp.float32)
        c_blk = lax.dot_general(onehot, jnp.ones((blk, 1), jnp.float32),
                                (((0,), (0,)), ((), ())),
                                preferred_element_type=jnp.float32)

        @pl.when(i == 0)
        def _():
            s_ref[...] = jnp.zeros_like(s_ref)
            c_ref[...] = jnp.zeros_like(c_ref)
        s_ref[...] += s_blk
        c_ref[...] += c_blk

    return pl.pallas_call(
        body,
        grid=(N_PAD // blk,),
        in_specs=[pl.BlockSpec((blk, H), lambda i: (i, 0)),
                  pl.BlockSpec((blk, 1), lambda i: (i, 0))],
        out_specs=[pl.BlockSpec((n_b, H), lambda i: (0, 0)),
                   pl.BlockSpec((n_b, 1), lambda i: (0, 0))],
        out_shape=[jax.ShapeDtypeStruct((n_b, H), jnp.float32),
                   jax.ShapeDtypeStruct((n_b, 1), jnp.float32)],
    )(h, batch_f)


def _head(s, cnt_b, add8, w0gT, w0aT, b0, w1T, b1, w2T, b2c, w3T, b3,
          lng, lnb, n_b):
    def ln(z, g, b):
        mu = jnp.mean(z, axis=-1, keepdims=True)
        var = jnp.mean((z - mu) ** 2, axis=-1, keepdims=True)
        return (z - mu) / jnp.sqrt(var + 1e-5) * g + b

    def body(s_r, c_r, a_r, w0g_r, w0a_r, b0_r, w1_r, b1_r, w2_r, b2_r,
             w3_r, b3_r, lng_r, lnb_r, o_ref):
        g = s_r[...] / jnp.maximum(c_r[...], 1.0)
        z = (jnp.dot(g, w0g_r[...], preferred_element_type=jnp.float32)
             + jnp.dot(a_r[...], w0a_r[...], preferred_element_type=jnp.float32)
             + b0_r[...])
        z = jnp.maximum(ln(z, lng_r[0, 0:1, :], lnb_r[0, 0:1, :]), 0.0)
        z = jnp.dot(z, w1_r[...], preferred_element_type=jnp.float32) + b1_r[...]
        z = jnp.maximum(ln(z, lng_r[1, 0:1, :], lnb_r[1, 0:1, :]), 0.0)
        z = jnp.dot(z, w2_r[...], preferred_element_type=jnp.float32) + b2_r[...]
        z = jnp.maximum(ln(z, lng_r[2, 0:1, :], lnb_r[2, 0:1, :]), 0.0)
        z = jnp.dot(z, w3_r[...], preferred_element_type=jnp.float32) + b3_r[...]
        o_ref[...] = jnp.maximum(z, 0.0) + jnp.log1p(jnp.exp(-jnp.abs(z)))

    full = lambda shape: pl.BlockSpec(shape, lambda: tuple(0 for _ in shape))
    return pl.pallas_call(
        body,
        in_specs=[full((n_b, H)), full((n_b, 1)), full((n_b, 8)),
                  full((H, H)), full((8, H)), full((1, H)),
                  full((H, H)), full((1, H)), full((H, H)), full((1, H)),
                  full((H, 8)), full((1, 8)), full((3, 1, H)), full((3, 1, H))],
        out_specs=full((n_b, 8)),
        out_shape=jax.ShapeDtypeStruct((n_b, 8), jnp.float32),
    )(s, cnt_b, add8, w0gT, w0aT, b0, w1T, b1, w2T, b2c, w3T, b3, lng, lnb)


def kernel(x, edge_index, edge_attr, batch, conc, temp, pco2, params):
    f32 = jnp.float32
    n_b = 128

    x_pad = jnp.pad(x, ((0, N_PAD - N_NODES), (0, 0)))
    src = jnp.pad(edge_index[0], (0, E_PAD - N_EDGES),
                  constant_values=N_PAD - 1)
    dst = jnp.pad(edge_index[1], (0, E_PAD - N_EDGES),
                  constant_values=N_PAD - 1)
    ea_pad = jnp.pad(edge_attr, ((0, E_PAD - N_EDGES), (0, 0)))
    batch_f = jnp.pad(batch.astype(f32).reshape(-1, 1),
                      ((0, N_PAD - N_NODES), (0, 0)), constant_values=-1.0)
    add8 = jnp.pad(jnp.stack([conc, temp, pco2], axis=1).astype(f32),
                   ((0, 0), (0, 5)))

    Ls = params['layers']
    WeT_all, ce0_all = [], []
    prep = []
    for i in range(3):
        p = Ls[i]
        w1a = p['msg_w1'][:, :H]
        w1b = p['msg_w1'][:, H:]
        We = w1b @ p['lin_edge_w']
        ce0 = w1b @ p['lin_edge_b'] + p['msg_b1']
        WeT_all.append(We.T.reshape(16, NC, HALF).transpose(1, 0, 2))
        ce0_all.append(ce0.reshape(NC, 1, HALF))
        prep.append(dict(
            lwT=p['lin_node_w'].T, lb=p['lin_node_b'],
            w1aT_split=w1a.T.reshape(H, NC, HALF).transpose(1, 0, 2),
            c0=(We.sum(axis=1) + ce0).reshape(1, H),
            w2T=p['msg_w2'].T, b2=p['msg_b2'].reshape(1, H),
            uw1aT=p['upd_w1'][:, :H].T, uw1bT=p['upd_w1'][:, H:].T,
            uw2T=p['upd_w2'].T, ub1=p['upd_b1'].reshape(1, H),
            ub2=p['upd_b2'].reshape(1, H),
            ln_g=params['norm_g'][i].reshape(1, H),
            ln_b=params['norm_b'][i].reshape(1, H),
        ))
    WeT_all = jnp.stack(WeT_all)
    ce0_all = jnp.stack(ce0_all)
    skips = jnp.maximum(params['skip_weights'], 0.0)

    E_all = _e_all(ea_pad, WeT_all, ce0_all)

    src2 = src.reshape(IDX_ROWS, ECHUNK)
    dst2 = dst.reshape(IDX_ROWS, ECHUNK)
    sc_edge = _make_sc_edge_kernel()
    cnt_sc = _make_sc_cnt_kernel()(dst)
    cnt_e = cnt_sc[:N_PAD, :1]

    h = x_pad
    for i in range(3):
        pr = prep[i]
        x_t = _mm_bias(h, pr['lwT'], pr['lb'])
        a_sp = _a_split(x_t, pr['w1aT_split'])
        a_tab = a_sp.reshape(NC * N_PAD, HALF)
        e_l = E_all[i].reshape(NC * E_PAD, HALF)
        r_sc = sc_edge(a_tab, e_l, src2, dst2)
        skip = (skips[i - 1].reshape(1, 1) if i > 0
                else jnp.zeros((1, 1), f32))
        h_prev = h if i > 0 else x_t
        h = _post_layer(
            r_sc[:N_PAD], r_sc[N_PAD:], a_sp[0], a_sp[1], x_t, cnt_e, h_prev,
            pr['w2T'], pr['b2'], pr['c0'], pr['uw1aT'], pr['uw1bT'],
            pr['uw2T'], pr['ub1'], pr['ub2'], pr['ln_g'], pr['ln_b'], skip)

    s, cnt_b = _pool(h, batch_f, n_b)

    fc = params['fc']
    out8 = _head(
        s, cnt_b, add8,
        fc['ws'][0][:, :H].T, jnp.pad(fc['ws'][0][:, H:], ((0, 0), (0, 5))).T,
        fc['bs'][0].reshape(1, H),
        fc['ws'][1].T, fc['bs'][1].reshape(1, H),
        fc['ws'][2].T, fc['bs'][2].reshape(1, H),
        jnp.pad(fc['ws'][3], ((0, 7), (0, 0))).T,
        jnp.pad(fc['bs'][3], (0, 7)).reshape(1, 8),
        jnp.stack([g.reshape(1, H) for g in fc['ln_g']]),
        jnp.stack([b.reshape(1, H) for b in fc['ln_b']]),
        n_b)
    return out8[:, :1]


# speedup vs baseline: 3.7333x; 1.4005x over previous
import functools

import jax
import jax.numpy as jnp
from jax import lax
from jax.experimental import pallas as pl
from jax.experimental.pallas import tpu as pltpu
from jax.experimental.pallas import tpu_sc as plsc

N_NODES = 10000
N_PAD = 10240
N_EDGES = 320000
E_PAD = 327680
ECHUNK = 64
ECHUNKS = 320
SUPER = 16
NSUPER = ECHUNKS // SUPER
IDX_ROWS = E_PAD // ECHUNK
CCHUNK = 128
CCHUNKS = E_PAD // (16 * CCHUNK)
ROWS_PER_TILE = 640
H = 256
HALF = 128
NC = 2
NS = 16


def _sc_mesh():
    return plsc.VectorSubcoreMesh(core_axis_name="c", subcore_axis_name="s",
                                  num_cores=NC, num_subcores=NS)


def _make_sc_edge_kernel():
    f32, i32 = jnp.float32, jnp.int32
    scratch = [
        pltpu.VMEM((ECHUNK, HALF), f32), pltpu.VMEM((ECHUNK, HALF), f32),
        pltpu.VMEM((ECHUNK, HALF), f32), pltpu.VMEM((ECHUNK, HALF), f32),
        pltpu.VMEM((SUPER, ECHUNK), i32), pltpu.VMEM((SUPER, ECHUNK), i32),
        pltpu.VMEM((SUPER, ECHUNK), i32), pltpu.VMEM((SUPER, ECHUNK), i32),
        pltpu.VMEM((ECHUNK,), i32), pltpu.VMEM((ECHUNK,), i32),
        pltpu.VMEM((ECHUNK,), i32), pltpu.VMEM((ECHUNK,), i32),
        pltpu.VMEM_SHARED((N_PAD, HALF), f32),
    ] + [pltpu.SemaphoreType.DMA] * 8

    def body(a_hbm, e_hbm, src2_hbm, dst2_hbm, r_out,
             e0, e1, g0, g1, ssb0, ssb1, dsb0, dsb1, sv0, sv1, dv0, dv1,
             acc, ldg0, ldg1, lde0, lde1, sc0, sc1, ix0, ix1):
        e_b, g_b = (e0, e1), (g0, g1)
        ssb, dsb = (ssb0, ssb1), (dsb0, dsb1)
        sv, dv = (sv0, sv1), (dv0, dv1)
        ldg, lde, scs, ixs = (ldg0, ldg1), (lde0, lde1), (sc0, sc1), (ix0, ix1)
        c = lax.axis_index("c")
        s = lax.axis_index("s")
        row_off = c * N_PAD
        zero16 = jnp.zeros((16,), jnp.float32)

        def zrow(r, _):
            for j in range(HALF // 16):
                g0[r, pl.ds(j * 16, 16)] = zero16
            return 0
        lax.fori_loop(0, ECHUNK, zrow, 0)
        for rep in range(ROWS_PER_TILE // ECHUNK):
            pltpu.sync_copy(
                g0, acc.at[pl.ds(s * ROWS_PER_TILE + rep * ECHUNK, ECHUNK)])
        plsc.subcore_barrier()

        def issue_idx(sbi, q):
            r0 = s * ECHUNKS + sbi * SUPER
            pltpu.async_copy(src2_hbm.at[pl.ds(r0, SUPER)], ssb[q], ixs[q])
            pltpu.async_copy(dst2_hbm.at[pl.ds(r0, SUPER)], dsb[q], ixs[q])

        def wait_idx(q):
            pltpu.make_async_copy(src2_hbm.at[pl.ds(0, SUPER)], ssb[q],
                                  ixs[q]).wait()
            pltpu.make_async_copy(dst2_hbm.at[pl.ds(0, SUPER)], dsb[q],
                                  ixs[q]).wait()

        def fill_regs(kk, my_sv, my_dv):
            row1 = kk & (SUPER - 1)
            q1 = (kk // SUPER) & 1
            for q in range(2):
                @pl.when(q1 == q)
                def _():
                    for i in range(ECHUNK // 16):
                        sl = pl.ds(i * 16, 16)
                        my_sv[sl] = ssb[q][row1, sl] + row_off
                        my_dv[sl] = dsb[q][row1, sl]

        def issue_chunk(kk, my_sv, my_e, my_g, my_ldg, my_lde):
            base = s * (ECHUNKS * ECHUNK) + kk * ECHUNK
            pltpu.async_copy(a_hbm.at[my_sv], my_g, my_ldg)
            pltpu.async_copy(e_hbm.at[pl.ds(c * E_PAD + base, ECHUNK)],
                             my_e, my_lde)

        issue_idx(0, 0)
        issue_idx(1, 1)
        wait_idx(0)
        fill_regs(0, sv0, dv0)
        issue_chunk(0, sv0, e0, g0, ldg0, lde0)

        def step(k, my_e, my_g, my_sv, my_dv, my_ldg, my_lde, my_sc,
                 nx_e, nx_g, nx_sv, nx_dv, nx_ldg, nx_lde, nx_sc):
            @pl.when(k < ECHUNKS - 1)
            def _():
                k1 = k + 1
                row1 = k1 & (SUPER - 1)
                sb1 = k1 // SUPER
                q1 = sb1 & 1

                @pl.when(row1 == 0)
                def _():
                    for q in range(2):
                        @pl.when(q1 == q)
                        def _():
                            wait_idx(q)
                            @pl.when(sb1 + 1 < NSUPER)
                            def _():
                                issue_idx(sb1 + 1, 1 - q)

                @pl.when(k >= 1)
                def _():
                    pltpu.make_async_copy(nx_g, acc.at[nx_dv], nx_sc).wait()
                fill_regs(k1, nx_sv, nx_dv)
                issue_chunk(k1, nx_sv, nx_e, nx_g, nx_ldg, nx_lde)

            pltpu.make_async_copy(a_hbm.at[my_sv], my_g, my_ldg).wait()
            pltpu.make_async_copy(e_hbm.at[pl.ds(0, ECHUNK)], my_e,
                                  my_lde).wait()

            def rowf(r, _):
                for j in range(HALF // 16):
                    sl = pl.ds(j * 16, 16)
                    my_g[r, sl] = jnp.maximum(my_g[r, sl] + my_e[r, sl], 0.0)
                return 0
            lax.fori_loop(0, ECHUNK, rowf, 0)

            pltpu.async_copy(my_g, acc.at[my_dv], my_sc, add=True)

        def loop(t, _):
            k = t * 2
            step(k, e0, g0, sv0, dv0, ldg0, lde0, sc0,
                 e1, g1, sv1, dv1, ldg1, lde1, sc1)
            step(k + 1, e1, g1, sv1, dv1, ldg1, lde1, sc1,
                 e0, g0, sv0, dv0, ldg0, lde0, sc0)
            return 0

        lax.fori_loop(0, ECHUNKS // 2, loop, 0)
        pltpu.make_async_copy(g0, acc.at[dv0], sc0).wait()
        pltpu.make_async_copy(g1, acc.at[dv1], sc1).wait()
        plsc.subcore_barrier()

        rb = s * ROWS_PER_TILE
        pltpu.sync_copy(acc.at[pl.ds(rb, ROWS_PER_TILE)],
                        r_out.at[pl.ds(c * N_PAD + rb, ROWS_PER_TILE)])

    return pl.kernel(
        body,
        out_type=jax.ShapeDtypeStruct((NC * N_PAD, HALF), jnp.float32),
        mesh=_sc_mesh(), scratch_types=scratch)


def _make_sc_cnt_kernel():
    scratch = [
        pltpu.VMEM((CCHUNK, HALF), jnp.float32),
        pltpu.VMEM((CCHUNK,), jnp.int32),
        pltpu.VMEM_SHARED((N_PAD, HALF), jnp.float32),
    ]

    def body(dst_hbm, cnt_out, cbuf, dst_v, cnt_acc):
        s = lax.axis_index("s")

        def fill(val):
            v = jnp.full((16,), val, jnp.float32)
            def frow(r, _):
                for j in range(HALF // 16):
                    cbuf[r, pl.ds(j * 16, 16)] = v
                return 0
            lax.fori_loop(0, CCHUNK, frow, 0)

        fill(0.0)
        for rep in range(ROWS_PER_TILE // CCHUNK):
            pltpu.sync_copy(
                cbuf, cnt_acc.at[pl.ds(s * ROWS_PER_TILE + rep * CCHUNK, CCHUNK)])
        fill(1.0)
        plsc.subcore_barrier()

        def chunk_body(k, _):
            base = pl.multiple_of(s * (CCHUNKS * CCHUNK) + k * CCHUNK, 8)
            pltpu.sync_copy(dst_hbm.at[pl.ds(base, CCHUNK)], dst_v)
            pltpu.sync_copy(cbuf, cnt_acc.at[dst_v], add=True)
            return 0

        lax.fori_loop(0, CCHUNKS, chunk_body, 0)
        plsc.subcore_barrier()

        c = lax.axis_index("c")
        rb = s * ROWS_PER_TILE
        pltpu.sync_copy(cnt_acc.at[pl.ds(rb, ROWS_PER_TILE)],
                        cnt_out.at[pl.ds(c * N_PAD + rb, ROWS_PER_TILE)])

    return pl.kernel(
        body,
        out_type=jax.ShapeDtypeStruct((NC * N_PAD, HALF), jnp.float32),
        mesh=_sc_mesh(), scratch_types=scratch)


def _mm_bias(x, wT, b):
    m, k = x.shape
    n = wT.shape[1]
    blk = 512
    b2 = b.reshape(1, n)

    def body(x_ref, w_ref, b_ref, o_ref):
        o_ref[...] = jnp.dot(x_ref[...], w_ref[...],
                             preferred_element_type=jnp.float32) + b_ref[...]

    return pl.pallas_call(
        body,
        grid=(m // blk,),
        in_specs=[pl.BlockSpec((blk, k), lambda i: (i, 0)),
                  pl.BlockSpec((k, n), lambda i: (0, 0)),
                  pl.BlockSpec((1, n), lambda i: (0, 0))],
        out_specs=pl.BlockSpec((blk, n), lambda i: (i, 0)),
        out_shape=jax.ShapeDtypeStruct((m, n), jnp.float32),
    )(x, wT, b2)


def _a_split(x_t, w1aT_split):
    blk = 512

    def body(x_ref, w_ref, o_ref):
        o_ref[0] = jnp.dot(x_ref[...], w_ref[0],
                           preferred_element_type=jnp.float32)

    return pl.pallas_call(
        body,
        grid=(NC, N_PAD // blk),
        in_specs=[pl.BlockSpec((blk, H), lambda c, i: (i, 0)),
                  pl.BlockSpec((1, H, HALF), lambda c, i: (c, 0, 0))],
        out_specs=pl.BlockSpec((1, blk, HALF), lambda c, i: (c, i, 0)),
        out_shape=jax.ShapeDtypeStruct((NC, N_PAD, HALF), jnp.float32),
    )(x_t, w1aT_split)


def _e_layer(ea_pad, WeT_c, ce0_c):
    blk = 4096
    nb = E_PAD // blk

    def body(ea_ref, w_ref, b_ref, o_ref):
        o_ref[...] = jnp.dot(ea_ref[...], w_ref[0],
                             preferred_element_type=jnp.float32) + b_ref[0]

    return pl.pallas_call(
        body,
        grid=(NC, nb),
        in_specs=[pl.BlockSpec((blk, 16), lambda c, i: (i, 0)),
                  pl.BlockSpec((1, 16, HALF), lambda c, i: (c, 0, 0)),
                  pl.BlockSpec((1, 1, HALF), lambda c, i: (c, 0, 0))],
        out_specs=pl.BlockSpec((blk, HALF), lambda c, i: (c * nb + i, 0)),
        out_shape=jax.ShapeDtypeStruct((NC * E_PAD, HALF), jnp.float32),
    )(ea_pad, WeT_c, ce0_c)


def _post_layer(r0, r1, a0, a1, x_t, cnt_e, h_prev, w2T, b2, c0,
                uw1aT, uw1bT, uw2T, ub1, ub2, ln_g, ln_b, skip):
    blk = 512

    def body(r0_r, r1_r, a0_r, a1_r, xt_r, cnt_r, hp_r, w2_r, b2_r, c0_r,
             u1a_r, u1b_r, u2_r, ub1_r, ub2_r, g_r, bln_r, sk_r, o_ref):
        r_full = jnp.concatenate([r0_r[...], r1_r[...]], axis=-1)
        a_full = jnp.concatenate([a0_r[...], a1_r[...]], axis=-1)
        r_full = r_full + jnp.maximum(a_full + c0_r[...], 0.0)
        cnt = cnt_r[...] + 1.0
        aggr = jnp.dot(r_full, w2_r[...],
                       preferred_element_type=jnp.float32) / cnt + b2_r[...]
        u = jnp.maximum(
            jnp.dot(aggr, u1a_r[...], preferred_element_type=jnp.float32)
            + jnp.dot(xt_r[...], u1b_r[...], preferred_element_type=jnp.float32)
            + ub1_r[...], 0.0)
        u = jnp.dot(u, u2_r[...], preferred_element_type=jnp.float32) + ub2_r[...]
        mu = jnp.mean(u, axis=-1, keepdims=True)
        var = jnp.mean((u - mu) ** 2, axis=-1, keepdims=True)
        u = (u - mu) / jnp.sqrt(var + 1e-5) * g_r[...] + bln_r[...]
        o_ref[...] = u + sk_r[...] * hp_r[...]

    full = lambda shape: pl.BlockSpec(shape, lambda i: (0, 0))
    return pl.pallas_call(
        body,
        grid=(N_PAD // blk,),
        in_specs=[pl.BlockSpec((blk, HALF), lambda i: (i, 0)),
                  pl.BlockSpec((blk, HALF), lambda i: (i, 0)),
                  pl.BlockSpec((blk, HALF), lambda i: (i, 0)),
                  pl.BlockSpec((blk, HALF), lambda i: (i, 0)),
                  pl.BlockSpec((blk, H), lambda i: (i, 0)),
                  pl.BlockSpec((blk, 1), lambda i: (i, 0)),
                  pl.BlockSpec((blk, H), lambda i: (i, 0)),
                  full((H, H)), full((1, H)), full((1, H)),
                  full((H, H)), full((H, H)), full((H, H)),
                  full((1, H)), full((1, H)), full((1, H)), full((1, H)),
                  full((1, 1))],
        out_specs=pl.BlockSpec((blk, H), lambda i: (i, 0)),
        out_shape=jax.ShapeDtypeStruct((N_PAD, H), jnp.float32),
    )(r0, r1, a0, a1, x_t, cnt_e, h_prev, w2T, b2, c0,
      uw1aT, uw1bT, uw2T, ub1, ub2, ln_g, ln_b, skip)


def _pool(h, batch_f, n_b):
    blk = 512

    def body(h_ref, b_ref, s_ref, c_ref):
        i = pl.program_id(0)
        iota = lax.broadcasted_iota(jnp.int32, (1, n_b), 1).astype(jnp.float32)
        onehot = (b_ref[...] == iota).astype(jnp.float32)
        s_blk = lax.dot_general(onehot, h_ref[...],
                                (((0,), (0,)), ((), ())),
                                preferred_element_type=jnp.float32)
        c_blk = lax.dot_general(onehot, jnp.ones((blk, 1), jnp.float32),
                                (((0,), (0,)), ((), ())),
                                preferred_element_type=jnp.float32)

        @pl.when(i == 0)
        def _():
            s_ref[...] = jnp.zeros_like(s_ref)
            c_ref[...] = jnp.zeros_like(c_ref)
        s_ref[...] += s_blk
        c_ref[...] += c_blk

    return pl.pallas_call(
        body,
        grid=(N_PAD // blk,),
        in_specs=[pl.BlockSpec((blk, H), lambda i: (i, 0)),
                  pl.BlockSpec((blk, 1), lambda i: (i, 0))],
        out_specs=[pl.BlockSpec((n_b, H), lambda i: (0, 0)),
                   pl.BlockSpec((n_b, 1), lambda i: (0, 0))],
        out_shape=[jax.ShapeDtypeStruct((n_b, H), jnp.float32),
                   jax.ShapeDtypeStruct((n_b, 1), jnp.float32)],
    )(h, batch_f)


def _head(s, cnt_b, add8, w0gT, w0aT, b0, w1T, b1, w2T, b2c, w3T, b3,
          lng, lnb, n_b):
    def ln(z, g, b):
        mu = jnp.mean(z, axis=-1, keepdims=True)
        var = jnp.mean((z - mu) ** 2, axis=-1, keepdims=True)
        return (z - mu) / jnp.sqrt(var + 1e-5) * g + b

    def body(s_r, c_r, a_r, w0g_r, w0a_r, b0_r, w1_r, b1_r, w2_r, b2_r,
             w3_r, b3_r, lng_r, lnb_r, o_ref):
        g = s_r[...] / jnp.maximum(c_r[...], 1.0)
        z = (jnp.dot(g, w0g_r[...], preferred_element_type=jnp.float32)
             + jnp.dot(a_r[...], w0a_r[...], preferred_element_type=jnp.float32)
             + b0_r[...])
        z = jnp.maximum(ln(z, lng_r[0, 0:1, :], lnb_r[0, 0:1, :]), 0.0)
        z = jnp.dot(z, w1_r[...], preferred_element_type=jnp.float32) + b1_r[...]
        z = jnp.maximum(ln(z, lng_r[1, 0:1, :], lnb_r[1, 0:1, :]), 0.0)
        z = jnp.dot(z, w2_r[...], preferred_element_type=jnp.float32) + b2_r[...]
        z = jnp.maximum(ln(z, lng_r[2, 0:1, :], lnb_r[2, 0:1, :]), 0.0)
        z = jnp.dot(z, w3_r[...], preferred_element_type=jnp.float32) + b3_r[...]
        o_ref[...] = jnp.maximum(z, 0.0) + jnp.log1p(jnp.exp(-jnp.abs(z)))

    full = lambda shape: pl.BlockSpec(shape, lambda: tuple(0 for _ in shape))
    return pl.pallas_call(
        body,
        in_specs=[full((n_b, H)), full((n_b, 1)), full((n_b, 8)),
                  full((H, H)), full((8, H)), full((1, H)),
                  full((H, H)), full((1, H)), full((H, H)), full((1, H)),
                  full((H, 8)), full((1, 8)), full((3, 1, H)), full((3, 1, H))],
        out_specs=full((n_b, 8)),
        out_shape=jax.ShapeDtypeStruct((n_b, 8), jnp.float32),
    )(s, cnt_b, add8, w0gT, w0aT, b0, w1T, b1, w2T, b2c, w3T, b3, lng, lnb)


def kernel(x, edge_index, edge_attr, batch, conc, temp, pco2, params):
    f32 = jnp.float32
    n_b = 128

    x_pad = jnp.pad(x, ((0, N_PAD - N_NODES), (0, 0)))
    src = jnp.pad(edge_index[0], (0, E_PAD - N_EDGES),
                  constant_values=N_PAD - 1)
    dst = jnp.pad(edge_index[1], (0, E_PAD - N_EDGES),
                  constant_values=N_PAD - 1)
    ea_pad = jnp.pad(edge_attr, ((0, E_PAD - N_EDGES), (0, 0)))
    batch_f = jnp.pad(batch.astype(f32).reshape(-1, 1),
                      ((0, N_PAD - N_NODES), (0, 0)), constant_values=-1.0)
    add8 = jnp.pad(jnp.stack([conc, temp, pco2], axis=1).astype(f32),
                   ((0, 0), (0, 5)))

    Ls = params['layers']
    WeT_all, ce0_all = [], []
    prep = []
    for i in range(3):
        p = Ls[i]
        w1a = p['msg_w1'][:, :H]
        w1b = p['msg_w1'][:, H:]
        We = w1b @ p['lin_edge_w']
        ce0 = w1b @ p['lin_edge_b'] + p['msg_b1']
        WeT_all.append(We.T.reshape(16, NC, HALF).transpose(1, 0, 2))
        ce0_all.append(ce0.reshape(NC, 1, HALF))
        prep.append(dict(
            lwT=p['lin_node_w'].T, lb=p['lin_node_b'],
            w1aT_split=w1a.T.reshape(H, NC, HALF).transpose(1, 0, 2),
            c0=(We.sum(axis=1) + ce0).reshape(1, H),
            w2T=p['msg_w2'].T, b2=p['msg_b2'].reshape(1, H),
            uw1aT=p['upd_w1'][:, :H].T, uw1bT=p['upd_w1'][:, H:].T,
            uw2T=p['upd_w2'].T, ub1=p['upd_b1'].reshape(1, H),
            ub2=p['upd_b2'].reshape(1, H),
            ln_g=params['norm_g'][i].reshape(1, H),
            ln_b=params['norm_b'][i].reshape(1, H),
        ))
    skips = jnp.maximum(params['skip_weights'], 0.0)

    E_ls = [_e_layer(ea_pad, WeT_all[i], ce0_all[i]) for i in range(3)]

    src2 = src.reshape(IDX_ROWS, ECHUNK)
    dst2 = dst.reshape(IDX_ROWS, ECHUNK)
    sc_edge = _make_sc_edge_kernel()
    cnt_sc = _make_sc_cnt_kernel()(dst)
    cnt_e = cnt_sc[:N_PAD, :1]

    h = x_pad
    for i in range(3):
        pr = prep[i]
        x_t = _mm_bias(h, pr['lwT'], pr['lb'])
        a_sp = _a_split(x_t, pr['w1aT_split'])
        a_tab = a_sp.reshape(NC * N_PAD, HALF)
        r_sc = sc_edge(a_tab, E_ls[i], src2, dst2)
        skip = (skips[i - 1].reshape(1, 1) if i > 0
                else jnp.zeros((1, 1), f32))
        h_prev = h if i > 0 else x_t
        h = _post_layer(
            r_sc[:N_PAD], r_sc[N_PAD:], a_sp[0], a_sp[1], x_t, cnt_e, h_prev,
            pr['w2T'], pr['b2'], pr['c0'], pr['uw1aT'], pr['uw1bT'],
            pr['uw2T'], pr['ub1'], pr['ub2'], pr['ln_g'], pr['ln_b'], skip)

    s, cnt_b = _pool(h, batch_f, n_b)

    fc = params['fc']
    out8 = _head(
        s, cnt_b, add8,
        fc['ws'][0][:, :H].T, jnp.pad(fc['ws'][0][:, H:], ((0, 0), (0, 5))).T,
        fc['bs'][0].reshape(1, H),
        fc['ws'][1].T, fc['bs'][1].reshape(1, H),
        fc['ws'][2].T, fc['bs'][2].reshape(1, H),
        jnp.pad(fc['ws'][3], ((0, 7), (0, 0))).T,
        jnp.pad(fc['bs'][3], (0, 7)).reshape(1, 8),
        jnp.stack([g.reshape(1, H) for g in fc['ln_g']]),
        jnp.stack([b.reshape(1, H) for b in fc['ln_b']]),
        n_b)
    return out8[:, :1]
